# baseline (speedup 1.0000x reference)
import functools

import jax
import jax.numpy as jnp
from jax import lax
from jax.experimental import pallas as pl
from jax.experimental.pallas import tpu as pltpu
from jax.experimental.pallas import tpu_sc as plsc

NUM_CORES = 2
NUM_SUBCORES = 16
NUM_WORKERS = NUM_CORES * NUM_SUBCORES

LANES = 16
CHUNK = 128
PACK = 8


def _make_gather(batch: int, dim: int, n_super: int):
    b_per_w = batch // NUM_WORKERS
    n_chunks = b_per_w // CHUNK
    n_groups = b_per_w // LANES
    mesh = plsc.VectorSubcoreMesh(core_axis_name="c", subcore_axis_name="s")

    @functools.partial(
        pl.kernel,
        mesh=mesh,
        out_type=jax.ShapeDtypeStruct((batch, dim), jnp.float32),
        scratch_types=[
            pltpu.VMEM((n_chunks, CHUNK), jnp.int32),
            pltpu.VMEM((n_chunks, CHUNK), jnp.int32),
            pltpu.VMEM((n_groups, LANES), jnp.int32),
            pltpu.VMEM((2, CHUNK, PACK * dim), jnp.float32),
            pltpu.VMEM((b_per_w, dim), jnp.float32),
            pltpu.SemaphoreType.DMA,
            pltpu.SemaphoreType.DMA,
        ],
        compiler_params=pltpu.CompilerParams(needs_layout_passes=False),
    )
    def gather_kernel(idx_hbm, table_hbm, out_hbm,
                      idx_v, sup_v, off_v, gath_v, out_v, sem0, sem1):
        wid = lax.axis_index("s") * NUM_CORES + lax.axis_index("c")
        base = wid * b_per_w
        sems = (sem0, sem1)
        pltpu.sync_copy(idx_hbm.at[wid], idx_v)
        for r in range(n_chunks):
            for c in range(CHUNK // LANES):
                v = idx_v[r, pl.ds(c * LANES, LANES)]
                sup_v[r, pl.ds(c * LANES, LANES)] = v >> 3
                off_v[r * (CHUNK // LANES) + c, :] = (v & 7) << 4

        def fire(r):
            return pltpu.async_copy(
                table_hbm.at[sup_v.at[r]], gath_v.at[r % 2], sems[r % 2]
            )

        groups_per_chunk = CHUNK // LANES

        def extract_chunk(r):
            buf = gath_v.at[r % 2]

            def extract(g, carry):
                local = lax.iota(jnp.int32, LANES) + g * LANES
                rows = local + r * CHUNK
                cb = off_v[r * groups_per_chunk + g, :]
                for c in range(dim):
                    vals = plsc.load_gather(buf, [local, cb + c])
                    plsc.store_scatter(
                        out_v, [rows, jnp.full((LANES,), c, jnp.int32)], vals
                    )
                return carry
            lax.fori_loop(0, groups_per_chunk, extract, 0)

        copies = [None, None]
        copies[0] = fire(0)
        for r in range(n_chunks):
            if r + 1 < n_chunks:
                copies[(r + 1) % 2] = fire(r + 1)
            copies[r % 2].wait()
            extract_chunk(r)
        pltpu.sync_copy(out_v, out_hbm.at[pl.ds(base, b_per_w)])

    return gather_kernel


def kernel(objective_idx, table):
    batch = objective_idx.shape[0]
    num_rows, dim = table.shape
    table128 = table.reshape(num_rows // PACK, PACK * dim)
    idx3 = objective_idx.astype(jnp.int32).reshape(
        NUM_WORKERS, batch // NUM_WORKERS // CHUNK, CHUNK
    )
    return _make_gather(batch, dim, num_rows // PACK)(idx3, table128)


# speedup vs baseline: 6.0816x; 6.0816x over previous
import functools

import jax
import jax.numpy as jnp
from jax import lax
from jax.experimental import pallas as pl
from jax.experimental.pallas import tpu as pltpu
from jax.experimental.pallas import tpu_sc as plsc

NUM_CORES = 2
NUM_SUBCORES = 16
NUM_WORKERS = NUM_CORES * NUM_SUBCORES

LANES = 16
TILE_W = 128
JR = 2


def _make_gather(batch: int, dim: int, vocab: int):
    b_per_w = batch // NUM_WORKERS
    n_groups = b_per_w // LANES
    mesh = plsc.VectorSubcoreMesh(core_axis_name="c", subcore_axis_name="s")

    @functools.partial(
        pl.kernel,
        mesh=mesh,
        out_type=jax.ShapeDtypeStruct((dim, batch), jnp.float32),
        scratch_types=[
            pltpu.VMEM((b_per_w,), jnp.int32),
            pltpu.VMEM((2, LANES, JR, 8, TILE_W), jnp.float32),
            pltpu.VMEM((dim, b_per_w), jnp.float32),
            pltpu.SemaphoreType.DMA,
            pltpu.SemaphoreType.DMA,
        ],
        compiler_params=pltpu.CompilerParams(needs_layout_passes=False),
    )
    def gather_kernel(idx_hbm, table_hbm, out_hbm,
                      idx_v, buf_v, out_v, sem0, sem1):
        wid = lax.axis_index("s") * NUM_CORES + lax.axis_index("c")
        base = wid * b_per_w
        sems = (sem0, sem1)
        pltpu.sync_copy(idx_hbm.at[wid], idx_v)
        my_idx = idx_v

        def fire(g, slot):
            idxv = my_idx[pl.ds(g * LANES, LANES)]
            for kk in range(LANES):
                col = (idxv[kk] >> 7) * TILE_W
                pltpu.async_copy(
                    table_hbm.at[:, :, pl.ds(col, TILE_W)],
                    buf_v.at[slot, kk],
                    sems[slot],
                )

        def drain(slot):
            for kk in range(LANES):
                pltpu.make_async_copy(
                    table_hbm.at[:, :, pl.ds(0, TILE_W)],
                    buf_v.at[slot, kk],
                    sems[slot],
                ).wait()

        def extract(g, slot):
            idxv = my_idx[pl.ds(g * LANES, LANES)]
            o = idxv & (TILE_W - 1)
            lanes = lax.iota(jnp.int32, LANES)
            for j in range(dim):
                vals = plsc.load_gather(
                    buf_v.at[slot],
                    [
                        lanes,
                        jnp.full((LANES,), j // 8, jnp.int32),
                        jnp.full((LANES,), j % 8, jnp.int32),
                        o,
                    ],
                )
                out_v[j, pl.ds(g * LANES, LANES)] = vals

        fire(0, 0)

        def body(h, carry):
            g0 = 2 * h
            fire(g0 + 1, 1)
            drain(0)
            extract(g0, 0)

            @pl.when(g0 + 2 < n_groups)
            def _():
                fire(g0 + 2, 0)

            drain(1)
            extract(g0 + 1, 1)
            return carry

        lax.fori_loop(0, n_groups // 2, body, 0)
        pltpu.sync_copy(out_v, out_hbm.at[:, pl.ds(base, b_per_w)])

    return gather_kernel


def kernel(objective_idx, table):
    batch = objective_idx.shape[0]
    vocab, dim = table.shape
    t3 = table.T.reshape(JR, dim // JR, vocab)
    idx2 = objective_idx.astype(jnp.int32).reshape(
        NUM_WORKERS, batch // NUM_WORKERS
    )
    out_t = _make_gather(batch, dim, vocab)(idx2, t3)
    return out_t.T


# speedup vs baseline: 6.1375x; 1.0092x over previous
import functools

import jax
import jax.numpy as jnp
from jax import lax
from jax.experimental import pallas as pl
from jax.experimental.pallas import tpu as pltpu
from jax.experimental.pallas import tpu_sc as plsc

NUM_CORES = 2
NUM_SUBCORES = 16
NUM_WORKERS = NUM_CORES * NUM_SUBCORES

LANES = 16
TILE_W = 128
JR = 2


def _make_gather(batch: int, dim: int, vocab: int):
    b_per_w = batch // NUM_WORKERS
    n_groups = b_per_w // LANES
    mesh = plsc.VectorSubcoreMesh(core_axis_name="c", subcore_axis_name="s")

    @functools.partial(
        pl.kernel,
        mesh=mesh,
        out_type=jax.ShapeDtypeStruct((dim, batch), jnp.float32),
        scratch_types=[
            pltpu.VMEM((b_per_w,), jnp.int32),
            pltpu.VMEM((2, LANES, JR, 8, TILE_W), jnp.float32),
            pltpu.VMEM((dim, b_per_w), jnp.float32),
            pltpu.SemaphoreType.DMA,
            pltpu.SemaphoreType.DMA,
        ],
        compiler_params=pltpu.CompilerParams(needs_layout_passes=False),
    )
    def gather_kernel(idx_hbm, table_hbm, out_hbm,
                      idx_v, buf_v, out_v, sem0, sem1):
        wid = lax.axis_index("s") * NUM_CORES + lax.axis_index("c")
        base = wid * b_per_w
        sems = (sem0, sem1)
        pltpu.sync_copy(idx_hbm.at[pl.ds(base, b_per_w)], idx_v)
        my_idx = idx_v

        def fire(g, slot):
            idxv = my_idx[pl.ds(g * LANES, LANES)]
            for kk in range(LANES):
                col = (idxv[kk] >> 7) * TILE_W
                pltpu.async_copy(
                    table_hbm.at[:, :, pl.ds(col, TILE_W)],
                    buf_v.at[slot, kk],
                    sems[slot],
                )

        def drain(slot):
            for kk in range(LANES):
                pltpu.make_async_copy(
                    table_hbm.at[:, :, pl.ds(0, TILE_W)],
                    buf_v.at[slot, kk],
                    sems[slot],
                ).wait()

        def extract(g, slot):
            idxv = my_idx[pl.ds(g * LANES, LANES)]
            o = idxv & (TILE_W - 1)
            lanes = lax.iota(jnp.int32, LANES)
            for j in range(dim):
                vals = plsc.load_gather(
                    buf_v.at[slot],
                    [
                        lanes,
                        jnp.full((LANES,), j // 8, jnp.int32),
                        jnp.full((LANES,), j % 8, jnp.int32),
                        o,
                    ],
                )
                out_v[j, pl.ds(g * LANES, LANES)] = vals

        fire(0, 0)

        def body(h, carry):
            g0 = 2 * h
            fire(g0 + 1, 1)
            drain(0)
            extract(g0, 0)

            @pl.when(g0 + 2 < n_groups)
            def _():
                fire(g0 + 2, 0)

            drain(1)
            extract(g0 + 1, 1)
            return carry

        lax.fori_loop(0, n_groups // 2, body, 0)
        pltpu.sync_copy(out_v, out_hbm.at[:, pl.ds(base, b_per_w)])

    return gather_kernel


def kernel(objective_idx, table):
    batch = objective_idx.shape[0]
    vocab, dim = table.shape
    t3 = table.T.reshape(JR, dim // JR, vocab)
    out_t = _make_gather(batch, dim, vocab)(
        objective_idx.astype(jnp.int32), t3
    )
    return out_t.T
